# baseline (speedup 1.0000x reference)
import functools

import jax
import jax.numpy as jnp
from jax import lax
from jax.experimental import pallas as pl
from jax.experimental.pallas import tpu as pltpu
from jax.experimental.pallas import tpu_sc as plsc

NUM_CLASSES = 1000
BATCH = 4096
DENSE = 4096
W0 = 1.0
W1 = 0.5

SC_ROWS = 2560
TC_ROWS = BATCH - SC_ROWS

NC = 2
NS = 16
LANES = 16
NW = NC * NS
BPW = SC_ROWS // NW
CHUNK = 4
NCHUNK = BPW // CHUNK
UNROLL = 8

_sc_mesh = plsc.VectorSubcoreMesh(
    core_axis_name="c", subcore_axis_name="s", num_cores=NC, num_subcores=NS)


@functools.partial(
    pl.kernel,
    out_type=jax.ShapeDtypeStruct((NW, LANES), jnp.float32),
    mesh=_sc_mesh,
    scratch_types=[
        pltpu.VMEM((NCHUNK, CHUNK), jnp.int32),
        pltpu.VMEM((2, CHUNK, DENSE), jnp.float32),
        pltpu.VMEM((2, CHUNK, DENSE), jnp.float32),
        pltpu.VMEM((LANES,), jnp.float32),
        pltpu.SemaphoreType.DMA,
        pltpu.SemaphoreType.DMA,
        pltpu.SemaphoreType.DMA,
        pltpu.SemaphoreType.DMA,
    ],
)
def _sc_mse(o1_hbm, tgt_hbm, tab_hbm, out_hbm, idx_v, d_v, o_v, acc_v,
            sg0, sg1, sl0, sl1):
    wid = lax.axis_index("s") * NC + lax.axis_index("c")
    base = wid * BPW
    sg = (sg0, sg1)
    sl = (sl0, sl1)
    pltpu.sync_copy(tgt_hbm.at[wid], idx_v)

    def issue(ci, b):
        pltpu.async_copy(tab_hbm.at[idx_v.at[ci]], d_v.at[b], sg[b])
        pltpu.async_copy(o1_hbm.at[pl.ds(base + ci * CHUNK, CHUNK)],
                         o_v.at[b], sl[b])

    def wait(ci, b):
        pltpu.make_async_copy(tab_hbm.at[idx_v.at[ci]], d_v.at[b],
                              sg[b]).wait()
        pltpu.make_async_copy(o1_hbm.at[pl.ds(base + ci * CHUNK, CHUNK)],
                              o_v.at[b], sl[b]).wait()

    def compute(b, acc):
        for r in range(CHUNK):
            def vec_body(j, a):
                for u in range(UNROLL):
                    off = j * (LANES * UNROLL) + u * LANES
                    t = (o_v[b, r, pl.ds(off, LANES)]
                         - d_v[b, r, pl.ds(off, LANES)])
                    a = a + t * t
                return a
            acc = lax.fori_loop(0, DENSE // (LANES * UNROLL), vec_body, acc)
        return acc

    issue(0, 0)

    def pair_body(g, acc):
        ci0 = 2 * g
        ci1 = 2 * g + 1
        issue(ci1, 1)
        wait(ci0, 0)
        acc = compute(0, acc)
        nxt = jnp.minimum(ci0 + 2, NCHUNK - 1)
        issue(nxt, 0)
        wait(ci1, 1)
        acc = compute(1, acc)
        return acc

    acc = lax.fori_loop(0, NCHUNK // 2, pair_body,
                        jnp.zeros((LANES,), jnp.float32))
    wait(NCHUNK - 1, 0)
    acc_v[...] = acc
    pltpu.sync_copy(acc_v, out_hbm.at[wid])


CE_BLK = 1024
CE_GRID = BATCH // CE_BLK


def _ce_body(o0t_ref, tgt_ref, out_ref):
    x = o0t_ref[...]
    tgt = tgt_ref[...]
    m = jnp.max(x, axis=0, keepdims=True)
    e = jnp.exp(x - m)
    s = jnp.sum(e, axis=0, keepdims=True)
    lse = jnp.log(s) + m
    cls = lax.broadcasted_iota(jnp.int32, (NUM_CLASSES, CE_BLK), 0)
    onehot = (cls == tgt).astype(jnp.float32)
    tsum = jnp.sum(x * onehot)
    nll_sum = jnp.sum(lse) - tsum

    @pl.when(pl.program_id(0) == 0)
    def _():
        out_ref[...] = jnp.zeros_like(out_ref)

    out_ref[...] += jnp.reshape(nll_sum, (1, 1))


_tc_ce = pl.pallas_call(
    _ce_body,
    grid=(CE_GRID,),
    in_specs=[
        pl.BlockSpec((NUM_CLASSES, CE_BLK), lambda i: (0, i)),
        pl.BlockSpec((1, CE_BLK), lambda i: (0, i)),
    ],
    out_specs=pl.BlockSpec((1, 1), lambda i: (0, 0)),
    out_shape=jax.ShapeDtypeStruct((1, 1), jnp.float32),
)


MSE_BLK = 512
MSE_GRID = TC_ROWS // MSE_BLK


def _tc_mse_body(o1_ref, tgt_ref, lb_ref, out_ref):
    i = pl.program_id(0)

    @pl.when(i == 0)
    def _():
        out_ref[...] = jnp.zeros_like(out_ref)

    o1 = o1_ref[...]
    tgt = tgt_ref[...]
    cls = lax.broadcasted_iota(jnp.int32, (MSE_BLK, NUM_CLASSES), 1)
    onehot = (cls == tgt).astype(jnp.bfloat16)
    g = jnp.dot(onehot, lb_ref[...], preferred_element_type=jnp.float32)
    t = o1 - g
    out_ref[...] += jnp.reshape(jnp.sum(t * t), (1, 1))


_tc_mse = pl.pallas_call(
    _tc_mse_body,
    grid=(MSE_GRID,),
    in_specs=[
        pl.BlockSpec((MSE_BLK, DENSE),
                     lambda i: (SC_ROWS // MSE_BLK + i, 0)),
        pl.BlockSpec((MSE_BLK, 1), lambda i: (SC_ROWS // MSE_BLK + i, 0)),
        pl.BlockSpec((NUM_CLASSES, DENSE), lambda i: (0, 0)),
    ],
    out_specs=pl.BlockSpec((1, 1), lambda i: (0, 0)),
    out_shape=jax.ShapeDtypeStruct((1, 1), jnp.float32),
)


def kernel(output_0, output_1, target, dense_labels):
    tgt = target.astype(jnp.int32)
    tgt2d = tgt.reshape(BATCH, 1)
    sc_part = _sc_mse(output_1,
                      tgt[:SC_ROWS].reshape(NW, NCHUNK, CHUNK),
                      dense_labels)
    tc_part = _tc_mse(output_1, tgt2d, dense_labels.astype(jnp.bfloat16))
    ce_sum = _tc_ce(output_0.T, tgt.reshape(1, BATCH))
    mse = (jnp.sum(sc_part) + tc_part[0, 0]) * (1.0 / (BATCH * DENSE))
    ce = ce_sum[0, 0] * (1.0 / BATCH)
    return W0 * ce + W1 * mse


# speedup vs baseline: 1.0706x; 1.0706x over previous
import functools

import jax
import jax.numpy as jnp
from jax import lax
from jax.experimental import pallas as pl
from jax.experimental.pallas import tpu as pltpu
from jax.experimental.pallas import tpu_sc as plsc

NUM_CLASSES = 1000
BATCH = 4096
DENSE = 4096
W0 = 1.0
W1 = 0.5

SC_ROWS = 2560
TC_ROWS = BATCH - SC_ROWS

NC = 2
NS = 16
LANES = 16
NW = NC * NS
BPW = SC_ROWS // NW
CHUNK = 4
NCHUNK = BPW // CHUNK
UNROLL = 8

_sc_mesh = plsc.VectorSubcoreMesh(
    core_axis_name="c", subcore_axis_name="s", num_cores=NC, num_subcores=NS)


@functools.partial(
    pl.kernel,
    out_type=jax.ShapeDtypeStruct((NW, LANES), jnp.float32),
    mesh=_sc_mesh,
    scratch_types=[
        pltpu.VMEM((NCHUNK, CHUNK), jnp.int32),
        pltpu.VMEM((2, CHUNK, DENSE), jnp.float32),
        pltpu.VMEM((2, CHUNK, DENSE), jnp.float32),
        pltpu.VMEM((LANES,), jnp.float32),
        pltpu.SemaphoreType.DMA,
        pltpu.SemaphoreType.DMA,
        pltpu.SemaphoreType.DMA,
        pltpu.SemaphoreType.DMA,
    ],
)
def _sc_mse(o1_hbm, tgt_hbm, tab_hbm, out_hbm, idx_v, d_v, o_v, acc_v,
            sg0, sg1, sl0, sl1):
    wid = lax.axis_index("s") * NC + lax.axis_index("c")
    base = wid * BPW
    sg = (sg0, sg1)
    sl = (sl0, sl1)
    pltpu.sync_copy(tgt_hbm.at[wid], idx_v)

    def issue(ci, b):
        pltpu.async_copy(tab_hbm.at[idx_v.at[ci]], d_v.at[b], sg[b])
        pltpu.async_copy(o1_hbm.at[pl.ds(base + ci * CHUNK, CHUNK)],
                         o_v.at[b], sl[b])

    def wait(ci, b):
        pltpu.make_async_copy(tab_hbm.at[idx_v.at[ci]], d_v.at[b],
                              sg[b]).wait()
        pltpu.make_async_copy(o1_hbm.at[pl.ds(base + ci * CHUNK, CHUNK)],
                              o_v.at[b], sl[b]).wait()

    def compute(b, acc):
        for r in range(CHUNK):
            def vec_body(j, a):
                for u in range(UNROLL):
                    off = j * (LANES * UNROLL) + u * LANES
                    t = (o_v[b, r, pl.ds(off, LANES)]
                         - d_v[b, r, pl.ds(off, LANES)])
                    a = a + t * t
                return a
            acc = lax.fori_loop(0, DENSE // (LANES * UNROLL), vec_body, acc)
        return acc

    issue(0, 0)

    def pair_body(g, acc):
        ci0 = 2 * g
        ci1 = 2 * g + 1
        issue(ci1, 1)
        wait(ci0, 0)
        acc = compute(0, acc)
        nxt = jnp.minimum(ci0 + 2, NCHUNK - 1)
        issue(nxt, 0)
        wait(ci1, 1)
        acc = compute(1, acc)
        return acc

    acc = lax.fori_loop(0, NCHUNK // 2, pair_body,
                        jnp.zeros((LANES,), jnp.float32))
    wait(NCHUNK - 1, 0)
    acc_v[...] = acc
    pltpu.sync_copy(acc_v, out_hbm.at[wid])


CE_BLK = 1024
CE_GRID = BATCH // CE_BLK


def _ce_body(o0t_ref, tgt_ref, out_ref):
    x = o0t_ref[...]
    tgt = tgt_ref[...]
    m = jnp.max(x, axis=0, keepdims=True)
    e = jnp.exp(x - m)
    s = jnp.sum(e, axis=0, keepdims=True)
    lse = jnp.log(s) + m
    cls = lax.broadcasted_iota(jnp.int32, (NUM_CLASSES, CE_BLK), 0)
    onehot = (cls == tgt).astype(jnp.float32)
    tsum = jnp.sum(x * onehot)
    nll_sum = jnp.sum(lse) - tsum

    @pl.when(pl.program_id(0) == 0)
    def _():
        out_ref[...] = jnp.zeros_like(out_ref)

    out_ref[...] += jnp.reshape(nll_sum, (1, 1))


_tc_ce = pl.pallas_call(
    _ce_body,
    grid=(CE_GRID,),
    in_specs=[
        pl.BlockSpec((NUM_CLASSES, CE_BLK), lambda i: (0, i)),
        pl.BlockSpec((1, CE_BLK), lambda i: (0, i)),
    ],
    out_specs=pl.BlockSpec((1, 1), lambda i: (0, 0)),
    out_shape=jax.ShapeDtypeStruct((1, 1), jnp.float32),
)


MSE_BLK = 512
MSE_GRID = TC_ROWS // MSE_BLK


def _tc_mse_body(o1_ref, tgt_ref, lab_ref, out_ref, lb_ref):
    i = pl.program_id(0)

    @pl.when(i == 0)
    def _():
        out_ref[...] = jnp.zeros_like(out_ref)
        lb_ref[...] = lab_ref[...].astype(jnp.bfloat16)

    o1 = o1_ref[...]
    tgt = tgt_ref[...]
    cls = lax.broadcasted_iota(jnp.int32, (MSE_BLK, NUM_CLASSES), 1)
    onehot = (cls == tgt).astype(jnp.bfloat16)
    g = jnp.dot(onehot, lb_ref[...], preferred_element_type=jnp.float32)
    t = o1 - g
    out_ref[...] += jnp.reshape(jnp.sum(t * t), (1, 1))


_tc_mse = pl.pallas_call(
    _tc_mse_body,
    grid=(MSE_GRID,),
    in_specs=[
        pl.BlockSpec((MSE_BLK, DENSE),
                     lambda i: (SC_ROWS // MSE_BLK + i, 0)),
        pl.BlockSpec((MSE_BLK, 1), lambda i: (SC_ROWS // MSE_BLK + i, 0)),
        pl.BlockSpec((NUM_CLASSES, DENSE), lambda i: (0, 0)),
    ],
    out_specs=pl.BlockSpec((1, 1), lambda i: (0, 0)),
    out_shape=jax.ShapeDtypeStruct((1, 1), jnp.float32),
    scratch_shapes=[
        pltpu.VMEM((NUM_CLASSES, DENSE), jnp.bfloat16),
    ],
)


def kernel(output_0, output_1, target, dense_labels):
    tgt = target.astype(jnp.int32)
    tgt2d = tgt.reshape(BATCH, 1)
    sc_part = _sc_mse(output_1,
                      tgt[:SC_ROWS].reshape(NW, NCHUNK, CHUNK),
                      dense_labels)
    tc_part = _tc_mse(output_1, tgt2d, dense_labels)
    ce_sum = _tc_ce(output_0.T, tgt.reshape(1, BATCH))
    mse = (jnp.sum(sc_part) + tc_part[0, 0]) * (1.0 / (BATCH * DENSE))
    ce = ce_sum[0, 0] * (1.0 / BATCH)
    return W0 * ce + W1 * mse
